# baseline (speedup 1.0000x reference)
import jax
import jax.numpy as jnp
from jax import lax
from jax.experimental import pallas as pl
from jax.experimental.pallas import tpu as pltpu
from jax.experimental.pallas import tpu_sc as plsc

_f32 = jnp.float32
_CHUNK = 128
_NT = 16
_GROUP = 8


def _row_partition(N):
    rpt = ((N + _NT) // _NT + 7) // 8 * 8
    acc_rows = rpt * _NT
    full_tiles = N // rpt
    rem = N - full_tiles * rpt
    return rpt, acc_rows, full_tiles, rem


def _zero_rows_buf(rows, Fh):
    z16 = jnp.zeros((16,), _f32)

    def zrow(i, carry):
        for j in range(Fh // 16):
            rows[i, pl.ds(j * 16, 16)] = z16
        return carry

    lax.fori_loop(0, _CHUNK, zrow, 0)


def _zero_acc_slice(rows, acc, base, rpt):
    nfull, tail = rpt // _CHUNK, rpt % _CHUNK
    for k in range(nfull):
        pltpu.sync_copy(rows, acc.at[pl.ds(base + k * _CHUNK, _CHUNK)])
    if tail:
        pltpu.sync_copy(rows.at[pl.ds(0, tail)],
                        acc.at[pl.ds(base + nfull * _CHUNK, tail)])


def _writeout(pred, src_ref, dst_ref, base, n):
    @pl.when(pred)
    def _():
        pltpu.sync_copy(src_ref.at[pl.ds(base, n)],
                        dst_ref.at[pl.ds(base, n)])


def _edge_pass(h_ref, il3, acc, idxg, rows, sem, t_group0, n_groups,
               cacc=None, ones=None):
    def group(i, carry):
        pltpu.sync_copy(il3.at[t_group0 + i], idxg)
        for j in range(_GROUP):
            pltpu.async_copy(h_ref.at[idxg.at[2 * j]], rows, sem).wait()
            pltpu.sync_copy(rows, acc.at[idxg.at[2 * j + 1]], add=True)
            if cacc is not None:
                pltpu.sync_copy(ones, cacc.at[idxg.at[2 * j + 1]], add=True)
        return carry

    lax.fori_loop(0, n_groups, group, 0)


def _seg_sum0(N, E_pad, F):
    E_half = E_pad // 2
    per_tile = E_half // _NT
    n_groups = per_tile // _CHUNK // _GROUP
    rpt, acc_rows, full_tiles, rem = _row_partition(N)
    zlen = (rpt + 15) // 16 * 16

    mesh = plsc.VectorSubcoreMesh(core_axis_name="c", subcore_axis_name="s",
                                  num_cores=2, num_subcores=_NT)
    out_type = [jax.ShapeDtypeStruct((N, F), _f32),
                jax.ShapeDtypeStruct((N, F), _f32),
                jax.ShapeDtypeStruct((N,), _f32),
                jax.ShapeDtypeStruct((N,), _f32)]
    scratch = [
        pltpu.VMEM_SHARED((acc_rows, F), _f32),
        pltpu.VMEM_SHARED((acc_rows,), _f32),
        pltpu.VMEM((2 * _GROUP, _CHUNK), jnp.int32),
        pltpu.VMEM((_CHUNK, F), _f32),
        pltpu.VMEM((_CHUNK,), _f32),
        pltpu.VMEM((zlen,), _f32),
        pltpu.SemaphoreType.DMA,
    ]

    def body(h, il3, p0, p1, cnt0, cnt1, acc, cacc, idxg, rows, ones,
             zvec, sem):
        c = lax.axis_index("c")
        s = lax.axis_index("s")
        c0 = c == 0
        c1 = c == 1
        z16 = jnp.zeros((16,), _f32)

        _zero_rows_buf(rows, F)
        base = s * rpt
        _zero_acc_slice(rows, acc, base, rpt)
        for j in range(zlen // 16):
            zvec[pl.ds(j * 16, 16)] = z16
        for j in range(_CHUNK // 16):
            ones[pl.ds(j * 16, 16)] = jnp.ones((16,), _f32)
        pltpu.sync_copy(zvec.at[pl.ds(0, rpt)], cacc.at[pl.ds(base, rpt)])
        plsc.subcore_barrier()

        t_group0 = (c * E_half + s * per_tile) // _CHUNK // _GROUP
        _edge_pass(h, il3, acc, idxg, rows, sem, t_group0, n_groups,
                   cacc=cacc, ones=ones)
        plsc.subcore_barrier()

        def cnt_out_via_vmem(pred, cnt_ref, cbase, n):
            @pl.when(pred)
            def _():
                pltpu.sync_copy(cacc.at[pl.ds(cbase, n)],
                                zvec.at[pl.ds(0, n)])
                pltpu.sync_copy(zvec.at[pl.ds(0, n)],
                                cnt_ref.at[pl.ds(cbase, n)])

        full_p = s < full_tiles
        _writeout(jnp.logical_and(full_p, c0), acc, p0, base, rpt)
        cnt_out_via_vmem(jnp.logical_and(full_p, c0), cnt0, base, rpt)
        _writeout(jnp.logical_and(full_p, c1), acc, p1, base, rpt)
        cnt_out_via_vmem(jnp.logical_and(full_p, c1), cnt1, base, rpt)
        if rem:
            rem_p = s == full_tiles
            rbase = full_tiles * rpt
            _writeout(jnp.logical_and(rem_p, c0), acc, p0, rbase, rem)
            cnt_out_via_vmem(jnp.logical_and(rem_p, c0), cnt0, rbase, rem)
            _writeout(jnp.logical_and(rem_p, c1), acc, p1, rbase, rem)
            cnt_out_via_vmem(jnp.logical_and(rem_p, c1), cnt1, rbase, rem)

    return pl.kernel(body, out_type=out_type, mesh=mesh,
                     scratch_types=scratch)


def _seg_sum_half(N, E_pad, Fh):
    per_tile = E_pad // _NT
    n_groups = per_tile // _CHUNK // _GROUP
    rpt, acc_rows, full_tiles, rem = _row_partition(N)

    mesh = plsc.VectorSubcoreMesh(core_axis_name="c", subcore_axis_name="s",
                                  num_cores=2, num_subcores=_NT)
    out_type = [jax.ShapeDtypeStruct((N, Fh), _f32),
                jax.ShapeDtypeStruct((N, Fh), _f32)]
    scratch = [
        pltpu.VMEM_SHARED((acc_rows, Fh), _f32),
        pltpu.VMEM((2 * _GROUP, _CHUNK), jnp.int32),
        pltpu.VMEM((_CHUNK, Fh), _f32),
        pltpu.SemaphoreType.DMA,
    ]

    def body(h_a, h_b, il3, o_a, o_b, acc, idxg, rows, sem):
        c = lax.axis_index("c")
        s = lax.axis_index("s")
        c0 = c == 0
        c1 = c == 1

        _zero_rows_buf(rows, Fh)
        base = s * rpt
        _zero_acc_slice(rows, acc, base, rpt)
        plsc.subcore_barrier()

        t_group0 = s * (per_tile // _CHUNK // _GROUP)

        @pl.when(c0)
        def _():
            _edge_pass(h_a, il3, acc, idxg, rows, sem, t_group0, n_groups)

        @pl.when(c1)
        def _():
            _edge_pass(h_b, il3, acc, idxg, rows, sem, t_group0, n_groups)

        plsc.subcore_barrier()

        full_p = s < full_tiles
        _writeout(jnp.logical_and(full_p, c0), acc, o_a, base, rpt)
        _writeout(jnp.logical_and(full_p, c1), acc, o_b, base, rpt)
        if rem:
            rem_p = s == full_tiles
            rbase = full_tiles * rpt
            _writeout(jnp.logical_and(rem_p, c0), acc, o_a, rbase, rem)
            _writeout(jnp.logical_and(rem_p, c1), acc, o_b, rbase, rem)

    return pl.kernel(body, out_type=out_type, mesh=mesh,
                     scratch_types=scratch)


def _sage_linear0(p0, p1, c02, c12, x, W_l, W_r, b2):
    N, F = x.shape
    H = W_l.shape[1]
    Hh = H // 2
    BR = 256
    grid = (N + BR - 1) // BR

    def bodyfn(a0, a1, cn0, cn1, xx, wl, wr, bb, oa, ob):
        inv = 1.0 / jnp.maximum(cn0[...] + cn1[...], 1.0)
        acc = jnp.dot((a0[...] + a1[...]) * inv, wl[...],
                      preferred_element_type=_f32)
        acc = acc + jnp.dot(xx[...], wr[...], preferred_element_type=_f32)
        acc = jnp.maximum(acc + bb[...], 0.0)
        oa[...] = acc[:, :Hh]
        ob[...] = acc[:, Hh:]

    return pl.pallas_call(
        bodyfn,
        grid=(grid,),
        in_specs=[
            pl.BlockSpec((BR, F), lambda i: (i, 0)),
            pl.BlockSpec((BR, F), lambda i: (i, 0)),
            pl.BlockSpec((BR, 1), lambda i: (i, 0)),
            pl.BlockSpec((BR, 1), lambda i: (i, 0)),
            pl.BlockSpec((BR, F), lambda i: (i, 0)),
            pl.BlockSpec((F, H), lambda i: (0, 0)),
            pl.BlockSpec((F, H), lambda i: (0, 0)),
            pl.BlockSpec((1, H), lambda i: (0, 0)),
        ],
        out_specs=[pl.BlockSpec((BR, Hh), lambda i: (i, 0)),
                   pl.BlockSpec((BR, Hh), lambda i: (i, 0))],
        out_shape=[jax.ShapeDtypeStruct((N, Hh), _f32),
                   jax.ShapeDtypeStruct((N, Hh), _f32)],
    )(p0, p1, c02, c12, x, W_l, W_r, b2)


def _sage_linear(agg_a, agg_b, c02, c12, h_a, h_b, Wl_a, Wl_b, Wr_a, Wr_b,
                 b2, relu):
    N, Fa = agg_a.shape
    Fh = h_a.shape[1]
    H = Wl_a.shape[1]
    Hh = H // 2
    BR = 256
    grid = (N + BR - 1) // BR

    def bodyfn(aa, ab, cn0, cn1, ha, hb, wla, wlb, wra, wrb, bb, oa, ob):
        inv = 1.0 / jnp.maximum(cn0[...] + cn1[...], 1.0)
        acc = jnp.dot(aa[...] * inv, wla[...], preferred_element_type=_f32)
        acc = acc + jnp.dot(ab[...] * inv, wlb[...],
                            preferred_element_type=_f32)
        acc = acc + jnp.dot(ha[...], wra[...], preferred_element_type=_f32)
        acc = acc + jnp.dot(hb[...], wrb[...], preferred_element_type=_f32)
        acc = acc + bb[...]
        if relu:
            acc = jnp.maximum(acc, 0.0)
        oa[...] = acc[:, :Hh]
        ob[...] = acc[:, Hh:]

    return pl.pallas_call(
        bodyfn,
        grid=(grid,),
        in_specs=[
            pl.BlockSpec((BR, Fa), lambda i: (i, 0)),
            pl.BlockSpec((BR, Fa), lambda i: (i, 0)),
            pl.BlockSpec((BR, 1), lambda i: (i, 0)),
            pl.BlockSpec((BR, 1), lambda i: (i, 0)),
            pl.BlockSpec((BR, Fh), lambda i: (i, 0)),
            pl.BlockSpec((BR, Fh), lambda i: (i, 0)),
            pl.BlockSpec((Fa, H), lambda i: (0, 0)),
            pl.BlockSpec((Fa, H), lambda i: (0, 0)),
            pl.BlockSpec((Fh, H), lambda i: (0, 0)),
            pl.BlockSpec((Fh, H), lambda i: (0, 0)),
            pl.BlockSpec((1, H), lambda i: (0, 0)),
        ],
        out_specs=[pl.BlockSpec((BR, Hh), lambda i: (i, 0)),
                   pl.BlockSpec((BR, Hh), lambda i: (i, 0))],
        out_shape=[jax.ShapeDtypeStruct((N, Hh), _f32),
                   jax.ShapeDtypeStruct((N, Hh), _f32)],
    )(agg_a, agg_b, c02, c12, h_a, h_b, Wl_a, Wl_b, Wr_a, Wr_b, b2)


def _pool_classify(h_a, h_b, batch_r, Wc_p, bc_p, G):
    N, Hh = h_a.shape
    H = 2 * Hh
    CP = Wc_p.shape[1]
    BR = 256
    grid = (N + BR - 1) // BR

    def bodyfn(ha, hb, bt, wc, bc, o, gsum, gcnt):
        i = pl.program_id(0)

        @pl.when(i == 0)
        def _():
            gsum[...] = jnp.zeros_like(gsum)
            gcnt[...] = jnp.zeros_like(gcnt)

        rows_col = i * BR + lax.broadcasted_iota(jnp.int32, (BR, 1), 0)
        vmask = rows_col < N
        ham = jnp.where(vmask, ha[...], 0.0)
        hbm = jnp.where(vmask, hb[...], 0.0)
        rows_row = i * BR + lax.broadcasted_iota(jnp.int32, (1, BR), 1)
        gids = lax.broadcasted_iota(jnp.int32, (G, BR), 0)
        onehot = jnp.where((gids == bt[...]) & (rows_row < N), 1.0, 0.0)
        gsum[:, :Hh] = gsum[:, :Hh] + jnp.dot(onehot, ham,
                                              preferred_element_type=_f32)
        gsum[:, Hh:] = gsum[:, Hh:] + jnp.dot(onehot, hbm,
                                              preferred_element_type=_f32)
        gcnt[...] = gcnt[...] + jnp.sum(onehot, axis=1, keepdims=True)

        @pl.when(i == grid - 1)
        def _():
            g = gsum[...] / jnp.maximum(gcnt[...], 1.0)
            o[...] = jnp.dot(g, wc[...], preferred_element_type=_f32) + bc[...]

    return pl.pallas_call(
        bodyfn,
        grid=(grid,),
        in_specs=[
            pl.BlockSpec((BR, Hh), lambda i: (i, 0)),
            pl.BlockSpec((BR, Hh), lambda i: (i, 0)),
            pl.BlockSpec((1, BR), lambda i: (0, i)),
            pl.BlockSpec((H, CP), lambda i: (0, 0)),
            pl.BlockSpec((1, CP), lambda i: (0, 0)),
        ],
        out_specs=pl.BlockSpec((G, CP), lambda i: (0, 0)),
        out_shape=jax.ShapeDtypeStruct((G, CP), _f32),
        scratch_shapes=[pltpu.VMEM((G, H), _f32), pltpu.VMEM((G, 1), _f32)],
    )(h_a, h_b, batch_r, Wc_p, bc_p)


def kernel(x, edge_index, batch, W_l0, W_r0, b0, W_l1, W_r1, b1, W_l2, W_r2,
           b2, W_cls, b_cls):
    N, D = x.shape
    H = W_l0.shape[1]
    C = W_cls.shape[1]
    E = edge_index.shape[1]
    G = 64

    epg = 2 * _NT * _CHUNK * _GROUP
    E_pad = (E + epg - 1) // epg * epg
    src = edge_index[0]
    dst = edge_index[1]
    if E_pad > E:
        pad = E_pad - E
        src = jnp.concatenate([src, jnp.zeros((pad,), jnp.int32)])
        dst = jnp.concatenate([dst, jnp.full((pad,), N, jnp.int32)])
    il3 = jnp.stack([src.reshape(-1, _CHUNK), dst.reshape(-1, _CHUNK)],
                    axis=1).reshape(E_pad // _CHUNK // _GROUP,
                                    2 * _GROUP, _CHUNK)

    Hh = H // 2

    p0, p1, cnt0, cnt1 = _seg_sum0(N, E_pad, D)(x, il3)
    c02 = cnt0.reshape(N, 1)
    c12 = cnt1.reshape(N, 1)
    h1a, h1b = _sage_linear0(p0, p1, c02, c12, x, W_l0, W_r0,
                             b0.reshape(1, H))

    seg_h = _seg_sum_half(N, E_pad, Hh)
    a1a, a1b = seg_h(h1a, h1b, il3)
    h2a, h2b = _sage_linear(a1a, a1b, c02, c12, h1a, h1b,
                            W_l1[:Hh], W_l1[Hh:], W_r1[:Hh], W_r1[Hh:],
                            b1.reshape(1, H), True)

    a2a, a2b = seg_h(h2a, h2b, il3)
    h3a, h3b = _sage_linear(a2a, a2b, c02, c12, h2a, h2b,
                            W_l2[:Hh], W_l2[Hh:], W_r2[:Hh], W_r2[Hh:],
                            b2.reshape(1, H), False)

    CP = 128
    Wc_p = jnp.pad(W_cls, ((0, 0), (0, CP - C)))
    bc_p = jnp.pad(b_cls, (0, CP - C)).reshape(1, CP)
    outp = _pool_classify(h3a, h3b, batch.reshape(1, N), Wc_p, bc_p, G)
    return outp[:, :C]


# speedup vs baseline: 1.3380x; 1.3380x over previous
import jax
import jax.numpy as jnp
from jax import lax
from jax.experimental import pallas as pl
from jax.experimental.pallas import tpu as pltpu
from jax.experimental.pallas import tpu_sc as plsc

_f32 = jnp.float32
_CHUNK = 128
_NT = 16


def _row_partition(N):
    rpt = ((N + _NT) // _NT + 7) // 8 * 8
    acc_rows = rpt * _NT
    full_tiles = N // rpt
    rem = N - full_tiles * rpt
    return rpt, acc_rows, full_tiles, rem


def _zero_rows_buf(rows, Fh):
    z16 = jnp.zeros((16,), _f32)

    def zrow(i, carry):
        for j in range(Fh // 16):
            rows[i, pl.ds(j * 16, 16)] = z16
        return carry

    lax.fori_loop(0, _CHUNK, zrow, 0)


def _zero_acc_slice(rows, acc, base, rpt):
    nfull, tail = rpt // _CHUNK, rpt % _CHUNK
    for k in range(nfull):
        pltpu.sync_copy(rows, acc.at[pl.ds(base + k * _CHUNK, _CHUNK)])
    if tail:
        pltpu.sync_copy(rows.at[pl.ds(0, tail)],
                        acc.at[pl.ds(base + nfull * _CHUNK, tail)])


def _writeout(pred, src_ref, dst_ref, base, n):
    @pl.when(pred)
    def _():
        pltpu.sync_copy(src_ref.at[pl.ds(base, n)],
                        dst_ref.at[pl.ds(base, n)])


def _edge_pass(h_ref, src, dst, acc, idx_s, idx_d, rows, sem, t_base,
               n_chunks, cacc=None, ones=None):
    def chunk(i, carry):
        eb = t_base + i * _CHUNK
        pltpu.sync_copy(src.at[pl.ds(eb, _CHUNK)], idx_s)
        gd = pltpu.async_copy(h_ref.at[idx_s], rows, sem)
        pltpu.sync_copy(dst.at[pl.ds(eb, _CHUNK)], idx_d)
        gd.wait()
        pltpu.sync_copy(rows, acc.at[idx_d], add=True)
        if cacc is not None:
            pltpu.sync_copy(ones, cacc.at[idx_d], add=True)
        return carry

    lax.fori_loop(0, n_chunks, chunk, 0)


def _seg_sum0(N, E_pad, F):
    E_half = E_pad // 2
    per_tile = E_half // _NT
    n_chunks = per_tile // _CHUNK
    rpt, acc_rows, full_tiles, rem = _row_partition(N)
    zlen = (rpt + 15) // 16 * 16

    mesh = plsc.VectorSubcoreMesh(core_axis_name="c", subcore_axis_name="s",
                                  num_cores=2, num_subcores=_NT)
    out_type = [jax.ShapeDtypeStruct((N, F), _f32),
                jax.ShapeDtypeStruct((N, F), _f32),
                jax.ShapeDtypeStruct((N,), _f32),
                jax.ShapeDtypeStruct((N,), _f32)]
    scratch = [
        pltpu.VMEM_SHARED((acc_rows, F), _f32),
        pltpu.VMEM_SHARED((acc_rows,), _f32),
        pltpu.VMEM((_CHUNK,), jnp.int32),
        pltpu.VMEM((_CHUNK,), jnp.int32),
        pltpu.VMEM((_CHUNK, F), _f32),
        pltpu.VMEM((_CHUNK,), _f32),
        pltpu.VMEM((zlen,), _f32),
        pltpu.SemaphoreType.DMA,
    ]

    def body(h, src, dst, p0, p1, cnt0, cnt1, acc, cacc, idx_s, idx_d,
             rows, ones, zvec, sem):
        c = lax.axis_index("c")
        s = lax.axis_index("s")
        c0 = c == 0
        c1 = c == 1
        z16 = jnp.zeros((16,), _f32)

        _zero_rows_buf(rows, F)
        base = s * rpt
        _zero_acc_slice(rows, acc, base, rpt)
        for j in range(zlen // 16):
            zvec[pl.ds(j * 16, 16)] = z16
        for j in range(_CHUNK // 16):
            ones[pl.ds(j * 16, 16)] = jnp.ones((16,), _f32)
        pltpu.sync_copy(zvec.at[pl.ds(0, rpt)], cacc.at[pl.ds(base, rpt)])
        plsc.subcore_barrier()

        t_base = c * E_half + s * per_tile
        _edge_pass(h, src, dst, acc, idx_s, idx_d, rows, sem, t_base,
                   n_chunks, cacc=cacc, ones=ones)
        plsc.subcore_barrier()

        def cnt_out_via_vmem(pred, cnt_ref, cbase, n):
            @pl.when(pred)
            def _():
                pltpu.sync_copy(cacc.at[pl.ds(cbase, n)],
                                zvec.at[pl.ds(0, n)])
                pltpu.sync_copy(zvec.at[pl.ds(0, n)],
                                cnt_ref.at[pl.ds(cbase, n)])

        full_p = s < full_tiles
        _writeout(jnp.logical_and(full_p, c0), acc, p0, base, rpt)
        cnt_out_via_vmem(jnp.logical_and(full_p, c0), cnt0, base, rpt)
        _writeout(jnp.logical_and(full_p, c1), acc, p1, base, rpt)
        cnt_out_via_vmem(jnp.logical_and(full_p, c1), cnt1, base, rpt)
        if rem:
            rem_p = s == full_tiles
            rbase = full_tiles * rpt
            _writeout(jnp.logical_and(rem_p, c0), acc, p0, rbase, rem)
            cnt_out_via_vmem(jnp.logical_and(rem_p, c0), cnt0, rbase, rem)
            _writeout(jnp.logical_and(rem_p, c1), acc, p1, rbase, rem)
            cnt_out_via_vmem(jnp.logical_and(rem_p, c1), cnt1, rbase, rem)

    return pl.kernel(body, out_type=out_type, mesh=mesh,
                     scratch_types=scratch)


def _seg_sum_half(N, E_pad, Fh):
    per_tile = E_pad // _NT
    n_chunks = per_tile // _CHUNK
    rpt, acc_rows, full_tiles, rem = _row_partition(N)

    mesh = plsc.VectorSubcoreMesh(core_axis_name="c", subcore_axis_name="s",
                                  num_cores=2, num_subcores=_NT)
    out_type = [jax.ShapeDtypeStruct((N, Fh), _f32),
                jax.ShapeDtypeStruct((N, Fh), _f32)]
    scratch = [
        pltpu.VMEM_SHARED((acc_rows, Fh), _f32),
        pltpu.VMEM((_CHUNK,), jnp.int32),
        pltpu.VMEM((_CHUNK,), jnp.int32),
        pltpu.VMEM((_CHUNK, Fh), _f32),
        pltpu.SemaphoreType.DMA,
    ]

    def body(h_a, h_b, src, dst, o_a, o_b, acc, idx_s, idx_d, rows, sem):
        c = lax.axis_index("c")
        s = lax.axis_index("s")
        c0 = c == 0
        c1 = c == 1

        _zero_rows_buf(rows, Fh)
        base = s * rpt
        _zero_acc_slice(rows, acc, base, rpt)
        plsc.subcore_barrier()

        t_base = s * per_tile

        @pl.when(c0)
        def _():
            _edge_pass(h_a, src, dst, acc, idx_s, idx_d, rows, sem, t_base,
                       n_chunks)

        @pl.when(c1)
        def _():
            _edge_pass(h_b, src, dst, acc, idx_s, idx_d, rows, sem, t_base,
                       n_chunks)

        plsc.subcore_barrier()

        full_p = s < full_tiles
        _writeout(jnp.logical_and(full_p, c0), acc, o_a, base, rpt)
        _writeout(jnp.logical_and(full_p, c1), acc, o_b, base, rpt)
        if rem:
            rem_p = s == full_tiles
            rbase = full_tiles * rpt
            _writeout(jnp.logical_and(rem_p, c0), acc, o_a, rbase, rem)
            _writeout(jnp.logical_and(rem_p, c1), acc, o_b, rbase, rem)

    return pl.kernel(body, out_type=out_type, mesh=mesh,
                     scratch_types=scratch)


def _sage_linear0(p0, p1, c02, c12, x, W_l, W_r, b2):
    N, F = x.shape
    H = W_l.shape[1]
    Hh = H // 2
    BR = 256
    grid = (N + BR - 1) // BR

    def bodyfn(a0, a1, cn0, cn1, xx, wl, wr, bb, oa, ob):
        inv = 1.0 / jnp.maximum(cn0[...] + cn1[...], 1.0)
        acc = jnp.dot((a0[...] + a1[...]) * inv, wl[...],
                      preferred_element_type=_f32)
        acc = acc + jnp.dot(xx[...], wr[...], preferred_element_type=_f32)
        acc = jnp.maximum(acc + bb[...], 0.0)
        oa[...] = acc[:, :Hh]
        ob[...] = acc[:, Hh:]

    return pl.pallas_call(
        bodyfn,
        grid=(grid,),
        in_specs=[
            pl.BlockSpec((BR, F), lambda i: (i, 0)),
            pl.BlockSpec((BR, F), lambda i: (i, 0)),
            pl.BlockSpec((BR, 1), lambda i: (i, 0)),
            pl.BlockSpec((BR, 1), lambda i: (i, 0)),
            pl.BlockSpec((BR, F), lambda i: (i, 0)),
            pl.BlockSpec((F, H), lambda i: (0, 0)),
            pl.BlockSpec((F, H), lambda i: (0, 0)),
            pl.BlockSpec((1, H), lambda i: (0, 0)),
        ],
        out_specs=[pl.BlockSpec((BR, Hh), lambda i: (i, 0)),
                   pl.BlockSpec((BR, Hh), lambda i: (i, 0))],
        out_shape=[jax.ShapeDtypeStruct((N, Hh), _f32),
                   jax.ShapeDtypeStruct((N, Hh), _f32)],
    )(p0, p1, c02, c12, x, W_l, W_r, b2)


def _sage_linear(agg_a, agg_b, c02, c12, h_a, h_b, Wl_a, Wl_b, Wr_a, Wr_b,
                 b2, relu):
    N, Fa = agg_a.shape
    Fh = h_a.shape[1]
    H = Wl_a.shape[1]
    Hh = H // 2
    BR = 256
    grid = (N + BR - 1) // BR

    def bodyfn(aa, ab, cn0, cn1, ha, hb, wla, wlb, wra, wrb, bb, oa, ob):
        inv = 1.0 / jnp.maximum(cn0[...] + cn1[...], 1.0)
        acc = jnp.dot(aa[...] * inv, wla[...], preferred_element_type=_f32)
        acc = acc + jnp.dot(ab[...] * inv, wlb[...],
                            preferred_element_type=_f32)
        acc = acc + jnp.dot(ha[...], wra[...], preferred_element_type=_f32)
        acc = acc + jnp.dot(hb[...], wrb[...], preferred_element_type=_f32)
        acc = acc + bb[...]
        if relu:
            acc = jnp.maximum(acc, 0.0)
        oa[...] = acc[:, :Hh]
        ob[...] = acc[:, Hh:]

    return pl.pallas_call(
        bodyfn,
        grid=(grid,),
        in_specs=[
            pl.BlockSpec((BR, Fa), lambda i: (i, 0)),
            pl.BlockSpec((BR, Fa), lambda i: (i, 0)),
            pl.BlockSpec((BR, 1), lambda i: (i, 0)),
            pl.BlockSpec((BR, 1), lambda i: (i, 0)),
            pl.BlockSpec((BR, Fh), lambda i: (i, 0)),
            pl.BlockSpec((BR, Fh), lambda i: (i, 0)),
            pl.BlockSpec((Fa, H), lambda i: (0, 0)),
            pl.BlockSpec((Fa, H), lambda i: (0, 0)),
            pl.BlockSpec((Fh, H), lambda i: (0, 0)),
            pl.BlockSpec((Fh, H), lambda i: (0, 0)),
            pl.BlockSpec((1, H), lambda i: (0, 0)),
        ],
        out_specs=[pl.BlockSpec((BR, Hh), lambda i: (i, 0)),
                   pl.BlockSpec((BR, Hh), lambda i: (i, 0))],
        out_shape=[jax.ShapeDtypeStruct((N, Hh), _f32),
                   jax.ShapeDtypeStruct((N, Hh), _f32)],
    )(agg_a, agg_b, c02, c12, h_a, h_b, Wl_a, Wl_b, Wr_a, Wr_b, b2)


def _pool_classify(h_a, h_b, batch_r, Wc_p, bc_p, G):
    N, Hh = h_a.shape
    H = 2 * Hh
    CP = Wc_p.shape[1]
    BR = 256
    grid = (N + BR - 1) // BR

    def bodyfn(ha, hb, bt, wc, bc, o, gsum, gcnt):
        i = pl.program_id(0)

        @pl.when(i == 0)
        def _():
            gsum[...] = jnp.zeros_like(gsum)
            gcnt[...] = jnp.zeros_like(gcnt)

        rows_col = i * BR + lax.broadcasted_iota(jnp.int32, (BR, 1), 0)
        vmask = rows_col < N
        ham = jnp.where(vmask, ha[...], 0.0)
        hbm = jnp.where(vmask, hb[...], 0.0)
        rows_row = i * BR + lax.broadcasted_iota(jnp.int32, (1, BR), 1)
        gids = lax.broadcasted_iota(jnp.int32, (G, BR), 0)
        onehot = jnp.where((gids == bt[...]) & (rows_row < N), 1.0, 0.0)
        gsum[:, :Hh] = gsum[:, :Hh] + jnp.dot(onehot, ham,
                                              preferred_element_type=_f32)
        gsum[:, Hh:] = gsum[:, Hh:] + jnp.dot(onehot, hbm,
                                              preferred_element_type=_f32)
        gcnt[...] = gcnt[...] + jnp.sum(onehot, axis=1, keepdims=True)

        @pl.when(i == grid - 1)
        def _():
            g = gsum[...] / jnp.maximum(gcnt[...], 1.0)
            o[...] = jnp.dot(g, wc[...], preferred_element_type=_f32) + bc[...]

    return pl.pallas_call(
        bodyfn,
        grid=(grid,),
        in_specs=[
            pl.BlockSpec((BR, Hh), lambda i: (i, 0)),
            pl.BlockSpec((BR, Hh), lambda i: (i, 0)),
            pl.BlockSpec((1, BR), lambda i: (0, i)),
            pl.BlockSpec((H, CP), lambda i: (0, 0)),
            pl.BlockSpec((1, CP), lambda i: (0, 0)),
        ],
        out_specs=pl.BlockSpec((G, CP), lambda i: (0, 0)),
        out_shape=jax.ShapeDtypeStruct((G, CP), _f32),
        scratch_shapes=[pltpu.VMEM((G, H), _f32), pltpu.VMEM((G, 1), _f32)],
    )(h_a, h_b, batch_r, Wc_p, bc_p)


def kernel(x, edge_index, batch, W_l0, W_r0, b0, W_l1, W_r1, b1, W_l2, W_r2,
           b2, W_cls, b_cls):
    N, D = x.shape
    H = W_l0.shape[1]
    C = W_cls.shape[1]
    E = edge_index.shape[1]
    G = 64

    epg = 2 * _NT * _CHUNK
    E_pad = (E + epg - 1) // epg * epg
    src = edge_index[0]
    dst = edge_index[1]
    if E_pad > E:
        pad = E_pad - E
        src = jnp.concatenate([src, jnp.zeros((pad,), jnp.int32)])
        dst = jnp.concatenate([dst, jnp.full((pad,), N, jnp.int32)])

    Hh = H // 2

    p0, p1, cnt0, cnt1 = _seg_sum0(N, E_pad, D)(x, src, dst)
    c02 = cnt0.reshape(N, 1)
    c12 = cnt1.reshape(N, 1)
    h1a, h1b = _sage_linear0(p0, p1, c02, c12, x, W_l0, W_r0,
                             b0.reshape(1, H))

    seg_h = _seg_sum_half(N, E_pad, Hh)
    a1a, a1b = seg_h(h1a, h1b, src, dst)
    h2a, h2b = _sage_linear(a1a, a1b, c02, c12, h1a, h1b,
                            W_l1[:Hh], W_l1[Hh:], W_r1[:Hh], W_r1[Hh:],
                            b1.reshape(1, H), True)

    a2a, a2b = seg_h(h2a, h2b, src, dst)
    h3a, h3b = _sage_linear(a2a, a2b, c02, c12, h2a, h2b,
                            W_l2[:Hh], W_l2[Hh:], W_r2[:Hh], W_r2[Hh:],
                            b2.reshape(1, H), False)

    CP = 128
    Wc_p = jnp.pad(W_cls, ((0, 0), (0, CP - C)))
    bc_p = jnp.pad(b_cls, (0, CP - C)).reshape(1, CP)
    outp = _pool_classify(h3a, h3b, batch.reshape(1, N), Wc_p, bc_p, G)
    return outp[:, :C]
